# Initial kernel scaffold; baseline (speedup 1.0000x reference)
import jax
import jax.numpy as jnp
from jax.experimental import pallas as pl


def kernel(edges, edge_pairs, neigh, features, W_enc, weight):
    raise NotImplementedError("write your pallas kernel here")


# speedup vs baseline: 2.9790x; 2.9790x over previous
import functools

import jax
import jax.numpy as jnp
from jax import lax
from jax.experimental import pallas as pl
from jax.experimental.pallas import tpu as pltpu
from jax.experimental.pallas import tpu_sc as plsc

N_NODES = 10000
N_EDGES = 320000
D = 128
DEG = 32
B = 4096

NC = 2
NS = 16
NW = NC * NS
E_PER_W = B // NW
N_PER_W = 2 * E_PER_W
L = 16
VPD = D // L
NBUF = 2

_ROWS_BLK = 400


def _tc_body(f_ref, w1_ref, w2_ref, h1_ref, h2_ref):
    f = f_ref[...]
    dn = (((1,), (1,)), ((), ()))
    h1_ref[...] = lax.dot_general(f, w1_ref[...], dn,
                                  preferred_element_type=jnp.float32)
    h2 = lax.dot_general(f, w2_ref[...], dn,
                         preferred_element_type=jnp.float32)
    h2_ref[...] = h2 * (1.0 / DEG)


def _tc_encode(features, w1, w2):
    grid = (N_NODES // _ROWS_BLK,)
    return pl.pallas_call(
        _tc_body,
        grid=grid,
        in_specs=[
            pl.BlockSpec((_ROWS_BLK, D), lambda i: (i, 0)),
            pl.BlockSpec((D, D), lambda i: (0, 0)),
            pl.BlockSpec((D, D), lambda i: (0, 0)),
        ],
        out_specs=[
            pl.BlockSpec((_ROWS_BLK, D), lambda i: (i, 0)),
            pl.BlockSpec((_ROWS_BLK, D), lambda i: (i, 0)),
        ],
        out_shape=[
            jax.ShapeDtypeStruct((N_NODES, D), jnp.float32),
            jax.ShapeDtypeStruct((N_NODES, D), jnp.float32),
        ],
    )(features, w1, w2)


def _sc_body(edges_hbm, pairs_hbm, neigh_hbm, h1_hbm, h2_hbm, w_hbm, out_hbm,
             edges_v, idx_v, node_v, nrow_st, neigh_v,
             sfeat_v, bufs, wv_v, sc_v, out_v,
             sem_p, sem_n, sem_s, sem0, sem1):
    sems = (sem0, sem1)
    wid = lax.axis_index("s") * NC + lax.axis_index("c")
    base_e = wid * E_PER_W

    pltpu.sync_copy(edges_hbm.at[pl.ds(base_e, E_PER_W)], edges_v)

    for blk in range(E_PER_W // L):
        e = edges_v[pl.ds(blk * L, L)]
        idx_v[pl.ds(blk * L, L)] = e >> 6
    pltpu.async_copy(pairs_hbm.at[idx_v.at[pl.ds(0, E_PER_W)]],
                     nrow_st.at[pl.ds(0, E_PER_W)], sem_p).wait()
    for blk in range(N_PER_W // L):
        p = lax.iota(jnp.int32, L) + jnp.int32(blk * L)
        k = p >> 1
        e = plsc.load_gather(edges_v, [k])
        col = ((e & 63) << 1) + (p & 1)
        node_v[pl.ds(blk * L, L)] = plsc.load_gather(nrow_st, [k, col])

    for blk in range(N_PER_W // L):
        nv = node_v[pl.ds(blk * L, L)]
        idx_v[pl.ds(blk * L, L)] = nv >> 2
    cp_n = pltpu.async_copy(neigh_hbm.at[idx_v], nrow_st, sem_n)
    cp_s = pltpu.async_copy(h1_hbm.at[node_v], sfeat_v, sem_s)
    pltpu.sync_copy(w_hbm, wv_v)
    cp_n.wait()

    def _flat(i, _):
        for sub in range(8):
            base = i * 8 * L + sub * L
            p = lax.iota(jnp.int32, L) + base
            n = p >> 5
            j = p & 31
            nv = plsc.load_gather(node_v, [n])
            col = ((nv & 3) << 5) + j
            neigh_v[pl.ds(base, L)] = plsc.load_gather(nrow_st, [n, col])
        return _
    lax.fori_loop(0, N_PER_W * DEG // (8 * L), _flat, None)

    for b in range(NBUF):
        pltpu.async_copy(h2_hbm.at[neigh_v.at[pl.ds(b * DEG, DEG)]],
                         bufs.at[b], sems[b])
    cp_s.wait()

    w_regs = [wv_v[pl.ds(v * L, L)] for v in range(VPD)]

    def _chunk(i, _):
        for b in range(NBUF):
            n = i * NBUF + b
            pltpu.make_async_copy(
                h2_hbm.at[neigh_v.at[pl.ds(n * DEG, DEG)]],
                bufs.at[b], sems[b]).wait()
            acc = [bufs[b, 0, pl.ds(v * L, L)] for v in range(VPD)]
            for j in range(1, DEG):
                for v in range(VPD):
                    acc[v] = acc[v] + bufs[b, j, pl.ds(v * L, L)]
            nn = n + NBUF

            @pl.when(nn < N_PER_W)
            def _issue():
                pltpu.async_copy(
                    h2_hbm.at[neigh_v.at[pl.ds(nn * DEG, DEG)]],
                    bufs.at[b], sems[b])

            r = None
            for v in range(VPD):
                z = jnp.maximum(acc[v] + sfeat_v[n, pl.ds(v * L, L)], 0.0)
                t = z * w_regs[v]
                r = t if r is None else r + t
            sc_v[n, pl.ds(0, L)] = r
        return _

    lax.fori_loop(0, N_PER_W // NBUF, _chunk, None)

    for blk in range(E_PER_W // L):
        p = lax.iota(jnp.int32, L) + jnp.int32(blk * L)
        acc = None
        for c in range(L):
            cv = jnp.full((L,), c, jnp.int32)
            t = (plsc.load_gather(sc_v, [p * 2, cv])
                 + plsc.load_gather(sc_v, [p * 2 + 1, cv]))
            acc = t if acc is None else acc + t
        out_v[pl.ds(blk * L, L)] = acc * 0.5

    pltpu.sync_copy(out_v, out_hbm.at[pl.ds(base_e, E_PER_W)])


@functools.cache
def _sc_gather_fn():
  return pl.kernel(
    _sc_body,
    out_type=jax.ShapeDtypeStruct((B,), jnp.float32),
    mesh=plsc.VectorSubcoreMesh(core_axis_name="c", subcore_axis_name="s",
                                num_cores=NC, num_subcores=NS),
    compiler_params=pltpu.CompilerParams(needs_layout_passes=False),
    scratch_types=[
        pltpu.VMEM((E_PER_W,), jnp.int32),
        pltpu.VMEM((N_PER_W,), jnp.int32),
        pltpu.VMEM((N_PER_W,), jnp.int32),
        pltpu.VMEM((N_PER_W, 128), jnp.int32),
        pltpu.VMEM((N_PER_W * DEG,), jnp.int32),
        pltpu.VMEM((N_PER_W, D), jnp.float32),
        pltpu.VMEM((NBUF, DEG, D), jnp.float32),
        pltpu.VMEM((D,), jnp.float32),
        pltpu.VMEM((N_PER_W, L), jnp.float32),
        pltpu.VMEM((E_PER_W,), jnp.float32),
        pltpu.SemaphoreType.DMA,
        pltpu.SemaphoreType.DMA,
        pltpu.SemaphoreType.DMA,
        pltpu.SemaphoreType.DMA,
        pltpu.SemaphoreType.DMA,
    ],
  )


def kernel(edges, edge_pairs, neigh, features, W_enc, weight):
    w1 = W_enc[:, :D]
    w2 = W_enc[:, D:]
    h1, h2 = _tc_encode(features, w1, w2)
    pairs128 = edge_pairs.reshape(N_EDGES * 2 // 128, 128)
    neigh128 = neigh.reshape(N_NODES * DEG // 128, 128)
    scores = _sc_gather_fn()(edges, pairs128, neigh128, h1, h2,
                             weight.reshape(D))
    return scores.reshape(B, 1)


# speedup vs baseline: 5.8414x; 1.9608x over previous
import functools

import jax
import jax.numpy as jnp
from jax import lax
from jax.experimental import pallas as pl
from jax.experimental.pallas import tpu as pltpu
from jax.experimental.pallas import tpu_sc as plsc

N_NODES = 10000
N_EDGES = 320000
D = 128
DEG = 32
B = 4096

NC = 2
NS = 16
NW = NC * NS
E_PER_W = B // NW
N_PER_W = 2 * E_PER_W
L = 16
VPD = D // L
NBUF = 4

_ROWS_BLK = 400


def _tc_body(f_ref, w1_ref, w2_ref, h1_ref, h2_ref):
    f = f_ref[...]
    dn = (((1,), (1,)), ((), ()))
    h1_ref[...] = lax.dot_general(f, w1_ref[...], dn,
                                  preferred_element_type=jnp.float32)
    h2 = lax.dot_general(f, w2_ref[...], dn,
                         preferred_element_type=jnp.float32)
    h2_ref[...] = h2 * (1.0 / DEG)


def _tc_encode(features, w1, w2):
    grid = (N_NODES // _ROWS_BLK,)
    return pl.pallas_call(
        _tc_body,
        grid=grid,
        in_specs=[
            pl.BlockSpec((_ROWS_BLK, D), lambda i: (i, 0)),
            pl.BlockSpec((D, D), lambda i: (0, 0)),
            pl.BlockSpec((D, D), lambda i: (0, 0)),
        ],
        out_specs=[
            pl.BlockSpec((_ROWS_BLK, D), lambda i: (i, 0)),
            pl.BlockSpec((_ROWS_BLK, D), lambda i: (i, 0)),
        ],
        out_shape=[
            jax.ShapeDtypeStruct((N_NODES, D), jnp.float32),
            jax.ShapeDtypeStruct((N_NODES, D), jnp.float32),
        ],
    )(features, w1, w2)


def _sc_body(node_hbm, nidx_hbm, h1_hbm, h2_hbm, w_hbm, out_hbm,
             node_v, neigh_v, sfeat_v, bufs, wv_v, sc_v, out_v,
             sem_s, sem0, sem1, sem2, sem3):
    sems = (sem0, sem1, sem2, sem3)
    wid = lax.axis_index("s") * NC + lax.axis_index("c")
    base_e = wid * E_PER_W
    base_n = wid * N_PER_W

    pltpu.sync_copy(node_hbm.at[pl.ds(base_n, N_PER_W)], node_v)
    pltpu.sync_copy(nidx_hbm.at[pl.ds(base_n * DEG, N_PER_W * DEG)], neigh_v)
    cp_s = pltpu.async_copy(h1_hbm.at[node_v], sfeat_v, sem_s)
    pltpu.sync_copy(w_hbm, wv_v)

    for b in range(NBUF):
        pltpu.async_copy(h2_hbm.at[neigh_v.at[pl.ds(b * DEG, DEG)]],
                         bufs.at[b], sems[b])
    cp_s.wait()

    w_regs = [wv_v[pl.ds(v * L, L)] for v in range(VPD)]

    def _chunk(i, _):
        for b in range(NBUF):
            n = i * NBUF + b
            pltpu.make_async_copy(
                h2_hbm.at[neigh_v.at[pl.ds(n * DEG, DEG)]],
                bufs.at[b], sems[b]).wait()
            acc = [bufs[b, 0, pl.ds(v * L, L)] for v in range(VPD)]
            for j in range(1, DEG):
                for v in range(VPD):
                    acc[v] = acc[v] + bufs[b, j, pl.ds(v * L, L)]
            nn = n + NBUF

            @pl.when(nn < N_PER_W)
            def _issue():
                pltpu.async_copy(
                    h2_hbm.at[neigh_v.at[pl.ds(nn * DEG, DEG)]],
                    bufs.at[b], sems[b])

            r = None
            for v in range(VPD):
                z = jnp.maximum(acc[v] + sfeat_v[n, pl.ds(v * L, L)], 0.0)
                t = z * w_regs[v]
                r = t if r is None else r + t
            sc_v[n, pl.ds(0, L)] = r
        return _

    lax.fori_loop(0, N_PER_W // NBUF, _chunk, None)

    for blk in range(E_PER_W // L):
        p = lax.iota(jnp.int32, L) + jnp.int32(blk * L)
        acc = None
        for c in range(L):
            cv = jnp.full((L,), c, jnp.int32)
            t = (plsc.load_gather(sc_v, [p * 2, cv])
                 + plsc.load_gather(sc_v, [p * 2 + 1, cv]))
            acc = t if acc is None else acc + t
        out_v[pl.ds(blk * L, L)] = acc * 0.5

    pltpu.sync_copy(out_v, out_hbm.at[pl.ds(base_e, E_PER_W)])


@functools.cache
def _sc_gather_fn():
  return pl.kernel(
    _sc_body,
    out_type=jax.ShapeDtypeStruct((B,), jnp.float32),
    mesh=plsc.VectorSubcoreMesh(core_axis_name="c", subcore_axis_name="s",
                                num_cores=NC, num_subcores=NS),
    compiler_params=pltpu.CompilerParams(needs_layout_passes=False),
    scratch_types=[
        pltpu.VMEM((N_PER_W,), jnp.int32),
        pltpu.VMEM((N_PER_W * DEG,), jnp.int32),
        pltpu.VMEM((N_PER_W, D), jnp.float32),
        pltpu.VMEM((NBUF, DEG, D), jnp.float32),
        pltpu.VMEM((D,), jnp.float32),
        pltpu.VMEM((N_PER_W, L), jnp.float32),
        pltpu.VMEM((E_PER_W,), jnp.float32),
        pltpu.SemaphoreType.DMA,
        pltpu.SemaphoreType.DMA,
        pltpu.SemaphoreType.DMA,
        pltpu.SemaphoreType.DMA,
        pltpu.SemaphoreType.DMA,
    ],
  )


def kernel(edges, edge_pairs, neigh, features, W_enc, weight):
    w1 = W_enc[:, :D]
    w2 = W_enc[:, D:]
    h1, h2 = _tc_encode(features, w1, w2)
    node_list = jnp.take(edge_pairs, edges, axis=0).reshape(-1)
    neigh_flat = jnp.take(neigh, node_list, axis=0).reshape(-1)
    scores = _sc_gather_fn()(node_list, neigh_flat, h1, h2, weight.reshape(D))
    return scores.reshape(B, 1)
